# baseline (speedup 1.0000x reference)
import functools

import jax
import jax.numpy as jnp
from jax import lax
from jax.experimental import pallas as pl
from jax.experimental.pallas import tpu as pltpu
from jax.experimental.pallas import tpu_sc as plsc

NUM_CLASSES = 10
FEAT_DIM = 128
TAU = 0.5
P_CUTOFF = 0.5
B_ULB = 16384
B_LB = 4096
NC, NS = 2, 16
NW = NC * NS
ACC_ROWS = 16
CHUNK = 128
ULB_PER_W = B_ULB // NW
LB_PER_W = B_LB // NW
N_CHUNKS = ULB_PER_W // CHUNK + 1


def _prep_body(lT_ref, ohT_ref, llbT_ref, l2T_ref,
               idx_ref, cnt_ref, ohT_o, llbT_o, l2T_o):
    lT = lT_ref[...]
    m = jnp.max(lT, axis=0, keepdims=True)
    e = jnp.exp(lT - m)
    s = jnp.sum(e, axis=0, keepdims=True)
    mask = (1.0 / s) > P_CUTOFF
    row = lax.broadcasted_iota(jnp.int32, lT.shape, 0)
    is_max = lT == m
    amax = jnp.min(jnp.where(is_max, row, NUM_CLASSES), axis=0, keepdims=True)
    idx_ref[...] = jnp.where(mask, amax, NUM_CLASSES)
    oh = jnp.where((row == amax) & mask, 1.0, 0.0)
    ohT = ohT_ref[...]
    cnt_ref[...] = (jnp.sum(oh, axis=1, keepdims=True)
                    + jnp.sum(ohT, axis=1, keepdims=True))
    ohT_o[...] = ohT
    llbT_o[...] = llbT_ref[...]
    l2T_o[...] = l2T_ref[...]


def _sc_scatter_echo_body(weak_hbm, idxu_hbm, lb_hbm, idxl_hbm, hard_hbm,
                          out_hbm, weak_o, hard_o, lb_o,
                          buf, idxv, zbuf, rsems, wsems, isems, ssems, shared):
    cid = lax.axis_index("c")
    sid = lax.axis_index("s")
    wid = cid * NS + sid
    zero = jnp.zeros((16,), jnp.float32)

    @pl.loop(0, ACC_ROWS)
    def _zrow(i):
        for j in range(FEAT_DIM // 16):
            zbuf[i, pl.ds(j * 16, 16)] = zero

    @pl.when(sid == 0)
    def _zero_acc():
        pltpu.sync_copy(zbuf, shared)

    plsc.subcore_barrier()

    ru = wid * ULB_PER_W
    rl = wid * LB_PER_W
    jobs = []
    for j in range(ULB_PER_W // CHUNK):
        jobs.append((weak_hbm.at[pl.ds(ru + j * CHUNK, CHUNK)],
                     weak_o.at[pl.ds(ru + j * CHUNK, CHUNK)],
                     idxu_hbm.at[pl.ds(ru + j * CHUNK, CHUNK)]))
        jobs.append((hard_hbm.at[pl.ds(ru + j * CHUNK, CHUNK)],
                     hard_o.at[pl.ds(ru + j * CHUNK, CHUNK)], None))
    jobs.append((lb_hbm.at[pl.ds(rl, CHUNK)], lb_o.at[pl.ds(rl, CHUNK)],
                 idxl_hbm.at[pl.ds(rl, CHUNK)]))
    n = len(jobs)

    def issue_read(k):
        r = pltpu.async_copy(jobs[k][0], buf.at[k % 4], rsems.at[k % 4])
        ri = None
        if jobs[k][2] is not None:
            o = k // 2
            ri = pltpu.async_copy(jobs[k][2], idxv.at[o % 2],
                                  isems.at[o % 2])
        return r, ri

    rs = [issue_read(0), issue_read(1)]
    ws = [None] * n
    ss = []
    for k in range(n):
        b = k % 4
        r, ri = rs[k]
        r.wait()
        if ri is not None:
            ri.wait()
            o = k // 2
            ss.append(pltpu.async_copy(
                buf.at[b], shared.at[idxv.at[o % 2]], ssems.at[o % 2],
                add=True))
        ws[k] = pltpu.async_copy(buf.at[b], jobs[k][1], wsems.at[b])
        if k + 2 < n:
            if k - 2 >= 0:
                ws[k - 2].wait()
                if jobs[k - 2][2] is not None:
                    ss[(k - 2) // 2].wait()
            rs.append(issue_read(k + 2))
    ws[n - 4].wait()
    ws[n - 3].wait()
    ws[n - 2].wait()
    ws[n - 1].wait()
    ss[-1].wait()
    ss[-2].wait()

    plsc.subcore_barrier()

    @pl.when(sid == 0)
    def _writeback():
        pltpu.sync_copy(shared, zbuf)
        pltpu.sync_copy(zbuf, out_hbm.at[cid])


def _atten_body(w_ref, part_ref, cnt_ref, out_ref, pn_ref):
    @pl.when(pl.program_id(0) == 0)
    def _proto():
        psum = part_ref[0, :NUM_CLASSES, :] + part_ref[1, :NUM_CLASSES, :]
        p = psum / cnt_ref[...]
        pn_ref[...] = p / jnp.maximum(
            jnp.sqrt(jnp.sum(p * p, axis=1, keepdims=True)), 1e-12)

    p = pn_ref[...]
    w = w_ref[...]
    ssT = lax.dot_general(
        jnp.ones((1, FEAT_DIM), jnp.float32), w * w,
        (((1,), (1,)), ((), ())), preferred_element_type=jnp.float32)
    inv = (1.0 / TAU) / jnp.maximum(jnp.sqrt(ssT), 1e-12)
    lT = lax.dot_general(
        p, w, (((1,), (1,)), ((), ())),
        preferred_element_type=jnp.float32) * inv
    m = jnp.max(lT, axis=0, keepdims=True)
    e = jnp.exp(lT - m)
    out_ref[...] = e * (1.0 / jnp.sum(e, axis=0, keepdims=True))


def kernel(weak_feat, hard_feat, lb_feat, lb_one_hot, logits_x_lb,
           logits_x_ulb_1, logits_x_ulb_2, y_lb, y_ulb):
    idx_row, class_num, ohT_o, llbT_o, l2T_o = pl.pallas_call(
        _prep_body,
        out_shape=[
            jax.ShapeDtypeStruct((1, B_ULB), jnp.int32),
            jax.ShapeDtypeStruct((NUM_CLASSES, 1), jnp.float32),
            jax.ShapeDtypeStruct((NUM_CLASSES, B_LB), jnp.float32),
            jax.ShapeDtypeStruct((NUM_CLASSES, B_LB), jnp.float32),
            jax.ShapeDtypeStruct((NUM_CLASSES, B_ULB), jnp.float32),
        ],
    )(logits_x_ulb_1.T, lb_one_hot.T, logits_x_lb.T, logits_x_ulb_2.T)

    idx_ulb = idx_row.reshape(B_ULB)
    idx_lb = y_lb.astype(jnp.int32).reshape(B_LB)

    mesh = plsc.VectorSubcoreMesh(
        core_axis_name="c", subcore_axis_name="s",
        num_cores=NC, num_subcores=NS)

    sc_scatter_echo = functools.partial(
        pl.kernel,
        out_type=(
            jax.ShapeDtypeStruct((NC, ACC_ROWS, FEAT_DIM), jnp.float32),
            jax.ShapeDtypeStruct((B_ULB, FEAT_DIM), jnp.float32),
            jax.ShapeDtypeStruct((B_ULB, FEAT_DIM), jnp.float32),
            jax.ShapeDtypeStruct((B_LB, FEAT_DIM), jnp.float32),
        ),
        mesh=mesh,
        scratch_types=[
            pltpu.VMEM((4, CHUNK, FEAT_DIM), jnp.float32),
            pltpu.VMEM((2, CHUNK), jnp.int32),
            pltpu.VMEM((ACC_ROWS, FEAT_DIM), jnp.float32),
            pltpu.SemaphoreType.DMA((4,)),
            pltpu.SemaphoreType.DMA((4,)),
            pltpu.SemaphoreType.DMA((2,)),
            pltpu.SemaphoreType.DMA((2,)),
            pltpu.VMEM_SHARED((ACC_ROWS, FEAT_DIM), jnp.float32),
        ],
    )(_sc_scatter_echo_body)
    partials, weak_o, hard_o, lb_o = sc_scatter_echo(
        weak_feat, idx_ulb, lb_feat, idx_lb, hard_feat)

    n_blocks = 4
    blk = B_ULB // n_blocks
    agg_t = pl.pallas_call(
        _atten_body,
        grid=(n_blocks,),
        in_specs=[
            pl.BlockSpec((blk, FEAT_DIM), lambda i: (i, 0)),
            pl.BlockSpec((NC, ACC_ROWS, FEAT_DIM), lambda i: (0, 0, 0)),
            pl.BlockSpec((NUM_CLASSES, 1), lambda i: (0, 0)),
        ],
        out_specs=pl.BlockSpec((NUM_CLASSES, blk), lambda i: (0, i)),
        out_shape=jax.ShapeDtypeStruct((NUM_CLASSES, B_ULB), jnp.float32),
        scratch_shapes=[pltpu.VMEM((NUM_CLASSES, FEAT_DIM), jnp.float32)],
    )(weak_feat, partials, class_num)
    agg_out = agg_t.T

    return (weak_o, hard_o, lb_o, ohT_o.T, llbT_o.T, agg_out, l2T_o.T)
